# baseline (device time: 1263131 ns/iter reference)
import jax
import jax.numpy as jnp
from jax import lax
from jax.experimental import pallas as pl
from jax.experimental.pallas import tpu as pltpu

T = 4096
D = 2048
E = 4
F = 4096

Cp = 576
C = 2 * Cp
BF = 512


def _peer():
    return (lax.axis_index("x"), 1 - lax.axis_index("y"),
            lax.axis_index("z"))


def _neighbor_barrier():
    barrier = pltpu.get_barrier_semaphore()
    pl.semaphore_signal(barrier, inc=1, device_id=_peer(),
                        device_id_type=pl.DeviceIdType.MESH)
    pl.semaphore_wait(barrier, 1)


def _route_exchange(idx_clamped, x_bf):
    NR = 8 * Cp
    x_bf = x_bf.reshape(T, D // 128, 128)

    def body(idx_ref, x_ref, xg_ref, xr_ref, copy_sem, send_sem, recv_sem):
        def start(i, c):
            pltpu.make_async_copy(
                x_ref.at[idx_ref[i]], xg_ref.at[i], copy_sem).start()
            return c

        lax.fori_loop(0, NR, start, 0)

        def wait(i, c):
            pltpu.make_async_copy(
                x_ref.at[idx_ref[i]], xg_ref.at[i], copy_sem).wait()
            return c

        lax.fori_loop(0, NR, wait, 0)

        _neighbor_barrier()
        rdma = pltpu.make_async_remote_copy(
            src_ref=xg_ref.at[pl.ds(E * Cp, E * Cp)],
            dst_ref=xr_ref,
            send_sem=send_sem,
            recv_sem=recv_sem,
            device_id=_peer(),
            device_id_type=pl.DeviceIdType.MESH,
        )
        rdma.start()
        rdma.wait()

    xg, xr = pl.pallas_call(
        body,
        grid_spec=pltpu.PrefetchScalarGridSpec(
            num_scalar_prefetch=1,
            grid=(1,),
            in_specs=[pl.BlockSpec(memory_space=pl.ANY)],
            out_specs=[pl.BlockSpec(memory_space=pl.ANY),
                       pl.BlockSpec(memory_space=pl.ANY)],
            scratch_shapes=[
                pltpu.SemaphoreType.DMA,
                pltpu.SemaphoreType.DMA,
                pltpu.SemaphoreType.DMA,
            ],
        ),
        out_shape=[
            jax.ShapeDtypeStruct((NR, D // 128, 128), jnp.bfloat16),
            jax.ShapeDtypeStruct((E * Cp, D // 128, 128), jnp.bfloat16),
        ],
        compiler_params=pltpu.CompilerParams(collective_id=0),
    )(idx_clamped, x_bf)
    return xg.reshape(NR, D), xr.reshape(E * Cp, D)


def _exchange(arrays, collective_id):
    n = len(arrays)

    def body(*refs):
        in_refs = refs[:n]
        out_refs = refs[n:2 * n]
        send_sems, recv_sems = refs[2 * n], refs[2 * n + 1]

        _neighbor_barrier()
        rdmas = []
        for i in range(n):
            rdma = pltpu.make_async_remote_copy(
                src_ref=in_refs[i],
                dst_ref=out_refs[i],
                send_sem=send_sems.at[i],
                recv_sem=recv_sems.at[i],
                device_id=_peer(),
                device_id_type=pl.DeviceIdType.MESH,
            )
            rdma.start()
            rdmas.append(rdma)
        for rdma in rdmas:
            rdma.wait()

    return pl.pallas_call(
        body,
        out_shape=[jax.ShapeDtypeStruct(a.shape, a.dtype) for a in arrays],
        in_specs=[pl.BlockSpec(memory_space=pl.ANY)] * n,
        out_specs=[pl.BlockSpec(memory_space=pl.ANY)] * n,
        scratch_shapes=[
            pltpu.SemaphoreType.DMA((n,)),
            pltpu.SemaphoreType.DMA((n,)),
        ],
        compiler_params=pltpu.CompilerParams(collective_id=collective_id),
    )(*arrays)


def _moe_routed(xe, W1, W2):

    def body(x_ref, w1_ref, w2_ref, o_ref):
        f = pl.program_id(1)

        @pl.when(f == 0)
        def _():
            o_ref[...] = jnp.zeros_like(o_ref)

        h = jnp.maximum(
            jnp.dot(x_ref[...], w1_ref[0].astype(jnp.bfloat16),
                    preferred_element_type=jnp.float32), 0.0)
        o_ref[...] += jnp.dot(h.astype(jnp.bfloat16),
                              w2_ref[0].astype(jnp.bfloat16),
                              preferred_element_type=jnp.float32)

    return pl.pallas_call(
        body,
        grid=(E, F // BF),
        out_shape=jax.ShapeDtypeStruct((E * C, D), jnp.float32),
        in_specs=[
            pl.BlockSpec((C, D), lambda j, f: (j, 0)),
            pl.BlockSpec((1, D, BF), lambda j, f: (j, 0, f)),
            pl.BlockSpec((1, BF, D), lambda j, f: (j, f, 0)),
        ],
        out_specs=pl.BlockSpec((C, D), lambda j, f: (j, 0)),
        compiler_params=pltpu.CompilerParams(
            vmem_limit_bytes=52 * 1024 * 1024),
    )(xe, W1, W2)


def kernel(x, assign, W1, W2):
    my_y = lax.axis_index("y")
    e0 = E * my_y
    p0 = E - e0

    iota = jnp.arange(T, dtype=jnp.int32)
    idx8 = jnp.stack(
        [jnp.sort(jnp.where(assign == e, iota, T))[:Cp] for e in range(8)])
    idx = jnp.concatenate([
        lax.dynamic_slice(idx8, (e0, 0), (E, Cp)),
        lax.dynamic_slice(idx8, (p0, 0), (E, Cp)),
    ])

    idx_clamped = jnp.minimum(idx.reshape(-1), T - 1)
    x_bf = x.astype(jnp.bfloat16)
    xg, x_recv = _route_exchange(idx_clamped, x_bf)

    xe = jnp.concatenate(
        [xg[:E * Cp].reshape(E, Cp, D), x_recv.reshape(E, Cp, D)],
        axis=1).reshape(E * C, D)
    ye = _moe_routed(xe, W1, W2).reshape(E, C, D)

    (y_recv,) = _exchange(
        [ye[:, Cp:].astype(jnp.bfloat16)], collective_id=1)

    idx_mine = idx[:E].reshape(-1)
    idx_peer = idx[E:].reshape(-1)
    out = jnp.zeros((T, D), jnp.bfloat16)
    out = out.at[idx_mine].set(
        ye[:, :Cp].reshape(-1, D).astype(jnp.bfloat16), mode="drop")
    out = out.at[idx_peer].set(y_recv.reshape(-1, D), mode="drop")
    return out.astype(jnp.float32)


# device time: 690523 ns/iter; 1.8292x vs baseline; 1.8292x over previous
import jax
import jax.numpy as jnp
from jax import lax
from jax.experimental import pallas as pl
from jax.experimental.pallas import tpu as pltpu

T = 4096
D = 2048
E = 4
F = 4096

Cp = 576
C = 2 * Cp
BF = 512

BMr = 2 * Cp
BT = 512
BMo = 1024
BS = 768


def _peer():
    return (lax.axis_index("x"), 1 - lax.axis_index("y"),
            lax.axis_index("z"))


def _neighbor_barrier():
    barrier = pltpu.get_barrier_semaphore()
    pl.semaphore_signal(barrier, inc=1, device_id=_peer(),
                        device_id_type=pl.DeviceIdType.MESH)
    pl.semaphore_wait(barrier, 1)


def _exchange(arrays, collective_id):
    n = len(arrays)

    def body(*refs):
        in_refs = refs[:n]
        out_refs = refs[n:2 * n]
        send_sems, recv_sems = refs[2 * n], refs[2 * n + 1]

        _neighbor_barrier()
        rdmas = []
        for i in range(n):
            rdma = pltpu.make_async_remote_copy(
                src_ref=in_refs[i],
                dst_ref=out_refs[i],
                send_sem=send_sems.at[i],
                recv_sem=recv_sems.at[i],
                device_id=_peer(),
                device_id_type=pl.DeviceIdType.MESH,
            )
            rdma.start()
            rdmas.append(rdma)
        for rdma in rdmas:
            rdma.wait()

    return pl.pallas_call(
        body,
        out_shape=[jax.ShapeDtypeStruct(a.shape, a.dtype) for a in arrays],
        in_specs=[pl.BlockSpec(memory_space=pl.ANY)] * n,
        out_specs=[pl.BlockSpec(memory_space=pl.ANY)] * n,
        scratch_shapes=[
            pltpu.SemaphoreType.DMA((n,)),
            pltpu.SemaphoreType.DMA((n,)),
        ],
        compiler_params=pltpu.CompilerParams(collective_id=collective_id),
    )(*arrays)


def _route_gather(idx2d, x_bf):

    def body(idx_ref, x_ref, o_ref):
        t = pl.program_id(1)

        @pl.when(t == 0)
        def _():
            o_ref[...] = jnp.zeros_like(o_ref)

        cols = lax.broadcasted_iota(jnp.int32, (BMr, BT), 1) + t * BT
        onehot = (cols == idx_ref[...]).astype(jnp.bfloat16)
        o_ref[...] += jnp.dot(
            onehot, x_ref[...],
            preferred_element_type=jnp.float32).astype(jnp.bfloat16)

    return pl.pallas_call(
        body,
        grid=(8 * Cp // BMr, T // BT),
        out_shape=jax.ShapeDtypeStruct((8 * Cp, D), jnp.bfloat16),
        in_specs=[
            pl.BlockSpec((BMr, 1), lambda m, t: (m, 0)),
            pl.BlockSpec((BT, D), lambda m, t: (t, 0)),
        ],
        out_specs=pl.BlockSpec((BMr, D), lambda m, t: (m, 0)),
        compiler_params=pltpu.CompilerParams(
            vmem_limit_bytes=48 * 1024 * 1024),
    )(idx2d, x_bf)


def _unroute(idx_mine2d, idx_peer2d, ya, yb):

    def body(im_ref, ip_ref, ya_ref, yb_ref, o_ref):
        m = pl.program_id(0)
        s = pl.program_id(1)

        @pl.when(s == 0)
        def _():
            o_ref[...] = jnp.zeros_like(o_ref)

        rows = lax.broadcasted_iota(jnp.int32, (BMo, BS), 0) + m * BMo
        qa = (rows == im_ref[...]).astype(jnp.bfloat16)
        qb = (rows == ip_ref[...]).astype(jnp.bfloat16)
        o_ref[...] += jnp.dot(qa, ya_ref[...],
                              preferred_element_type=jnp.float32)
        o_ref[...] += jnp.dot(qb, yb_ref[...],
                              preferred_element_type=jnp.float32)

    return pl.pallas_call(
        body,
        grid=(T // BMo, E * Cp // BS),
        out_shape=jax.ShapeDtypeStruct((T, D), jnp.float32),
        in_specs=[
            pl.BlockSpec((1, BS), lambda m, s: (0, s)),
            pl.BlockSpec((1, BS), lambda m, s: (0, s)),
            pl.BlockSpec((BS, D), lambda m, s: (s, 0)),
            pl.BlockSpec((BS, D), lambda m, s: (s, 0)),
        ],
        out_specs=pl.BlockSpec((BMo, D), lambda m, s: (m, 0)),
        compiler_params=pltpu.CompilerParams(
            vmem_limit_bytes=48 * 1024 * 1024),
    )(idx_mine2d, idx_peer2d, ya, yb)


def _moe_routed(xe, W1, W2):

    def body(x_ref, w1_ref, w2_ref, o_ref):
        f = pl.program_id(1)

        @pl.when(f == 0)
        def _():
            o_ref[...] = jnp.zeros_like(o_ref)

        h = jnp.maximum(
            jnp.dot(x_ref[...], w1_ref[0].astype(jnp.bfloat16),
                    preferred_element_type=jnp.float32), 0.0)
        o_ref[...] += jnp.dot(h.astype(jnp.bfloat16),
                              w2_ref[0].astype(jnp.bfloat16),
                              preferred_element_type=jnp.float32)

    return pl.pallas_call(
        body,
        grid=(E, F // BF),
        out_shape=jax.ShapeDtypeStruct((E * C, D), jnp.float32),
        in_specs=[
            pl.BlockSpec((C, D), lambda j, f: (j, 0)),
            pl.BlockSpec((1, D, BF), lambda j, f: (j, 0, f)),
            pl.BlockSpec((1, BF, D), lambda j, f: (j, f, 0)),
        ],
        out_specs=pl.BlockSpec((C, D), lambda j, f: (j, 0)),
        compiler_params=pltpu.CompilerParams(
            vmem_limit_bytes=52 * 1024 * 1024),
    )(xe, W1, W2)


def kernel(x, assign, W1, W2):
    my_y = lax.axis_index("y")
    e0 = E * my_y
    p0 = E - e0

    iota = jnp.arange(T, dtype=jnp.int32)
    idx8 = jnp.stack(
        [jnp.sort(jnp.where(assign == e, iota, T))[:Cp] for e in range(8)])
    idx = jnp.concatenate([
        lax.dynamic_slice(idx8, (e0, 0), (E, Cp)),
        lax.dynamic_slice(idx8, (p0, 0), (E, Cp)),
    ]).reshape(-1)

    x_bf = x.astype(jnp.bfloat16)
    xg = _route_gather(
        jnp.minimum(idx, T - 1).reshape(8 * Cp, 1), x_bf)

    (x_recv,) = _exchange([xg[E * Cp:]], collective_id=0)

    xe = jnp.concatenate(
        [xg[:E * Cp].reshape(E, Cp, D), x_recv.reshape(E, Cp, D)],
        axis=1).reshape(E * C, D)
    ye = _moe_routed(xe, W1, W2).reshape(E, C, D)

    (y_recv,) = _exchange(
        [ye[:, Cp:].astype(jnp.bfloat16)], collective_id=1)

    return _unroute(
        idx[:E * Cp].reshape(1, E * Cp),
        idx[E * Cp:].reshape(1, E * Cp),
        ye[:, :Cp].reshape(E * Cp, D).astype(jnp.bfloat16),
        y_recv.reshape(E * Cp, D),
    )


# device time: 527169 ns/iter; 2.3961x vs baseline; 1.3099x over previous
import jax
import jax.numpy as jnp
from jax import lax
from jax.experimental import pallas as pl
from jax.experimental.pallas import tpu as pltpu

T = 4096
D = 2048
E = 4
F = 4096

Cp = 576
C = 2 * Cp
BF = 512

BMr = 2 * Cp
BT = 512
BMo = 1024
BS = 768


def _peer():
    return (lax.axis_index("x"), 1 - lax.axis_index("y"),
            lax.axis_index("z"))


def _neighbor_barrier():
    barrier = pltpu.get_barrier_semaphore()
    pl.semaphore_signal(barrier, inc=1, device_id=_peer(),
                        device_id_type=pl.DeviceIdType.MESH)
    pl.semaphore_wait(barrier, 1)


def _exchange(arrays, collective_id):
    n = len(arrays)

    def body(*refs):
        in_refs = refs[:n]
        out_refs = refs[n:2 * n]
        send_sems, recv_sems = refs[2 * n], refs[2 * n + 1]

        _neighbor_barrier()
        rdmas = []
        for i in range(n):
            rdma = pltpu.make_async_remote_copy(
                src_ref=in_refs[i],
                dst_ref=out_refs[i],
                send_sem=send_sems.at[i],
                recv_sem=recv_sems.at[i],
                device_id=_peer(),
                device_id_type=pl.DeviceIdType.MESH,
            )
            rdma.start()
            rdmas.append(rdma)
        for rdma in rdmas:
            rdma.wait()

    return pl.pallas_call(
        body,
        out_shape=[jax.ShapeDtypeStruct(a.shape, a.dtype) for a in arrays],
        in_specs=[pl.BlockSpec(memory_space=pl.ANY)] * n,
        out_specs=[pl.BlockSpec(memory_space=pl.ANY)] * n,
        scratch_shapes=[
            pltpu.SemaphoreType.DMA((n,)),
            pltpu.SemaphoreType.DMA((n,)),
        ],
        compiler_params=pltpu.CompilerParams(collective_id=collective_id),
    )(*arrays)


def _route_gather(idx2d, x_bf):

    def body(idx_ref, x_ref, o_ref):
        t = pl.program_id(1)

        @pl.when(t == 0)
        def _():
            o_ref[...] = jnp.zeros_like(o_ref)

        cols = lax.broadcasted_iota(jnp.int32, (BMr, BT), 1) + t * BT
        onehot = (cols == idx_ref[...]).astype(jnp.bfloat16)
        o_ref[...] += jnp.dot(
            onehot, x_ref[...],
            preferred_element_type=jnp.float32).astype(jnp.bfloat16)

    return pl.pallas_call(
        body,
        grid=(8 * Cp // BMr, T // BT),
        out_shape=jax.ShapeDtypeStruct((8 * Cp, D), jnp.bfloat16),
        in_specs=[
            pl.BlockSpec((BMr, 1), lambda m, t: (m, 0)),
            pl.BlockSpec((BT, D), lambda m, t: (t, 0)),
        ],
        out_specs=pl.BlockSpec((BMr, D), lambda m, t: (m, 0)),
        compiler_params=pltpu.CompilerParams(
            vmem_limit_bytes=48 * 1024 * 1024),
    )(idx2d, x_bf)


def _unroute(idx_mine2d, idx_peer2d, ya, yb):

    def body(im_ref, ip_ref, ya_ref, yb_ref, o_ref):
        m = pl.program_id(0)
        s = pl.program_id(1)

        @pl.when(s == 0)
        def _():
            o_ref[...] = jnp.zeros_like(o_ref)

        rows = lax.broadcasted_iota(jnp.int32, (BMo, BS), 0) + m * BMo
        qa = (rows == im_ref[...]).astype(jnp.bfloat16)
        qb = (rows == ip_ref[...]).astype(jnp.bfloat16)
        o_ref[...] += jnp.dot(qa, ya_ref[...],
                              preferred_element_type=jnp.float32)
        o_ref[...] += jnp.dot(qb, yb_ref[...],
                              preferred_element_type=jnp.float32)

    return pl.pallas_call(
        body,
        grid=(T // BMo, E * Cp // BS),
        out_shape=jax.ShapeDtypeStruct((T, D), jnp.float32),
        in_specs=[
            pl.BlockSpec((1, BS), lambda m, s: (0, s)),
            pl.BlockSpec((1, BS), lambda m, s: (0, s)),
            pl.BlockSpec((BS, D), lambda m, s: (s, 0)),
            pl.BlockSpec((BS, D), lambda m, s: (s, 0)),
        ],
        out_specs=pl.BlockSpec((BMo, D), lambda m, s: (m, 0)),
        compiler_params=pltpu.CompilerParams(
            vmem_limit_bytes=48 * 1024 * 1024),
    )(idx_mine2d, idx_peer2d, ya, yb)


NFB = F // BF


def _moe_fused(xg, W1, W2):

    def body(xloc_ref, xg_ref, w1_ref, w2_ref, ye_ref, yret_ref,
             xrecv, ystage, xs_send, xs_recv, ys_send, ys_recv):
        h = pl.program_id(0)
        j = pl.program_id(1)
        f = pl.program_id(2)

        def x_rdma(jj):
            return pltpu.make_async_remote_copy(
                src_ref=xg_ref.at[pl.ds((E + jj) * Cp, Cp)],
                dst_ref=xrecv.at[pl.ds(jj * Cp, Cp)],
                send_sem=xs_send.at[jj],
                recv_sem=xs_recv.at[jj],
                device_id=_peer(),
                device_id_type=pl.DeviceIdType.MESH,
            )

        def y_rdma(jj):
            return pltpu.make_async_remote_copy(
                src_ref=ystage.at[pl.ds(jj * Cp, Cp)],
                dst_ref=yret_ref.at[pl.ds(jj * Cp, Cp)],
                send_sem=ys_send.at[jj],
                recv_sem=ys_recv.at[jj],
                device_id=_peer(),
                device_id_type=pl.DeviceIdType.MESH,
            )

        is_first = jnp.logical_and(
            h == 0, jnp.logical_and(j == 0, f == 0))

        @pl.when(is_first)
        def _():
            _neighbor_barrier()
            for jj in range(E):
                x_rdma(jj).start()

        @pl.when(f == 0)
        def _():
            ye_ref[...] = jnp.zeros_like(ye_ref)

        @pl.when(jnp.logical_and(h == 1, f == 0))
        def _():
            x_rdma(j).wait_recv()

        w1 = w1_ref[0].astype(jnp.bfloat16)
        w2 = w2_ref[0].astype(jnp.bfloat16)

        @pl.when(h == 0)
        def _():
            hm = jnp.maximum(
                jnp.dot(xloc_ref[...], w1,
                        preferred_element_type=jnp.float32), 0.0)
            ye_ref[...] += jnp.dot(hm.astype(jnp.bfloat16), w2,
                                   preferred_element_type=jnp.float32)

        @pl.when(h == 1)
        def _():
            xb = xrecv[pl.ds(j * Cp, Cp), :]
            hm = jnp.maximum(
                jnp.dot(xb, w1, preferred_element_type=jnp.float32), 0.0)
            ye_ref[...] += jnp.dot(hm.astype(jnp.bfloat16), w2,
                                   preferred_element_type=jnp.float32)

        @pl.when(jnp.logical_and(h == 1, f == NFB - 1))
        def _():
            ystage[pl.ds(j * Cp, Cp), :] = ye_ref[...].astype(jnp.bfloat16)
            y_rdma(j).start()

        is_last = jnp.logical_and(
            h == 1, jnp.logical_and(j == E - 1, f == NFB - 1))

        @pl.when(is_last)
        def _():
            for jj in range(E):
                x_rdma(jj).wait_send()
                y_rdma(jj).wait_send()
                y_rdma(jj).wait_recv()

    return pl.pallas_call(
        body,
        grid=(2, E, NFB),
        out_shape=[
            jax.ShapeDtypeStruct((2 * E * Cp, D), jnp.float32),
            jax.ShapeDtypeStruct((E * Cp, D), jnp.bfloat16),
        ],
        in_specs=[
            pl.BlockSpec((Cp, D), lambda h, j, f: (j, 0)),
            pl.BlockSpec(memory_space=pl.ANY),
            pl.BlockSpec((1, D, BF), lambda h, j, f: (j, 0, f)),
            pl.BlockSpec((1, BF, D), lambda h, j, f: (j, f, 0)),
        ],
        out_specs=[
            pl.BlockSpec((Cp, D), lambda h, j, f: (h * E + j, 0)),
            pl.BlockSpec(memory_space=pl.ANY),
        ],
        scratch_shapes=[
            pltpu.VMEM((E * Cp, D), jnp.bfloat16),
            pltpu.VMEM((E * Cp, D), jnp.bfloat16),
            pltpu.SemaphoreType.DMA((E,)),
            pltpu.SemaphoreType.DMA((E,)),
            pltpu.SemaphoreType.DMA((E,)),
            pltpu.SemaphoreType.DMA((E,)),
        ],
        compiler_params=pltpu.CompilerParams(
            collective_id=0,
            vmem_limit_bytes=56 * 1024 * 1024),
    )(xg, xg, W1, W2)


def kernel(x, assign, W1, W2):
    my_y = lax.axis_index("y")
    e0 = E * my_y
    p0 = E - e0

    iota = jnp.arange(T, dtype=jnp.int32)
    idx8 = jnp.stack(
        [jnp.sort(jnp.where(assign == e, iota, T))[:Cp] for e in range(8)])
    idx = jnp.concatenate([
        lax.dynamic_slice(idx8, (e0, 0), (E, Cp)),
        lax.dynamic_slice(idx8, (p0, 0), (E, Cp)),
    ]).reshape(-1)

    x_bf = x.astype(jnp.bfloat16)
    xg = _route_gather(
        jnp.minimum(idx, T - 1).reshape(8 * Cp, 1), x_bf)

    ye_all, y_ret = _moe_fused(xg, W1, W2)

    return _unroute(
        idx[:E * Cp].reshape(1, E * Cp),
        idx[E * Cp:].reshape(1, E * Cp),
        ye_all[:E * Cp].astype(jnp.bfloat16),
        y_ret,
    )


# device time: 473810 ns/iter; 2.6659x vs baseline; 1.1126x over previous
import jax
import jax.numpy as jnp
from jax import lax
from jax.experimental import pallas as pl
from jax.experimental.pallas import tpu as pltpu

T = 4096
D = 2048
E = 4
F = 4096

Cp = 576
C = 2 * Cp
BF = 512

BMr = 2 * Cp
BT = 512
BMo = 1024
BS = 768


def _peer():
    return (lax.axis_index("x"), 1 - lax.axis_index("y"),
            lax.axis_index("z"))


def _neighbor_barrier():
    barrier = pltpu.get_barrier_semaphore()
    pl.semaphore_signal(barrier, inc=1, device_id=_peer(),
                        device_id_type=pl.DeviceIdType.MESH)
    pl.semaphore_wait(barrier, 1)


def _exchange(arrays, collective_id):
    n = len(arrays)

    def body(*refs):
        in_refs = refs[:n]
        out_refs = refs[n:2 * n]
        send_sems, recv_sems = refs[2 * n], refs[2 * n + 1]

        _neighbor_barrier()
        rdmas = []
        for i in range(n):
            rdma = pltpu.make_async_remote_copy(
                src_ref=in_refs[i],
                dst_ref=out_refs[i],
                send_sem=send_sems.at[i],
                recv_sem=recv_sems.at[i],
                device_id=_peer(),
                device_id_type=pl.DeviceIdType.MESH,
            )
            rdma.start()
            rdmas.append(rdma)
        for rdma in rdmas:
            rdma.wait()

    return pl.pallas_call(
        body,
        out_shape=[jax.ShapeDtypeStruct(a.shape, a.dtype) for a in arrays],
        in_specs=[pl.BlockSpec(memory_space=pl.ANY)] * n,
        out_specs=[pl.BlockSpec(memory_space=pl.ANY)] * n,
        scratch_shapes=[
            pltpu.SemaphoreType.DMA((n,)),
            pltpu.SemaphoreType.DMA((n,)),
        ],
        compiler_params=pltpu.CompilerParams(collective_id=collective_id),
    )(*arrays)


def _route_gather(idx2d, x_bf):

    def body(idx_ref, x_ref, o_ref):
        t = pl.program_id(1)

        @pl.when(t == 0)
        def _():
            o_ref[...] = jnp.zeros_like(o_ref)

        cols = lax.broadcasted_iota(jnp.int32, (BMr, BT), 1) + t * BT
        onehot = (cols == idx_ref[...]).astype(jnp.bfloat16)
        o_ref[...] += jnp.dot(
            onehot, x_ref[...].astype(jnp.bfloat16),
            preferred_element_type=jnp.float32).astype(jnp.bfloat16)

    return pl.pallas_call(
        body,
        grid=(8 * Cp // BMr, T // BT),
        out_shape=jax.ShapeDtypeStruct((8 * Cp, D), jnp.bfloat16),
        in_specs=[
            pl.BlockSpec((BMr, 1), lambda m, t: (m, 0)),
            pl.BlockSpec((BT, D), lambda m, t: (t, 0)),
        ],
        out_specs=pl.BlockSpec((BMr, D), lambda m, t: (m, 0)),
        compiler_params=pltpu.CompilerParams(
            vmem_limit_bytes=48 * 1024 * 1024),
    )(idx2d, x_bf)


def _unroute(idx_mine2d, idx_peer2d, ya, yb):

    def body(im_ref, ip_ref, ya_ref, yb_ref, o_ref):
        m = pl.program_id(0)
        s = pl.program_id(1)

        @pl.when(s == 0)
        def _():
            o_ref[...] = jnp.zeros_like(o_ref)

        rows = lax.broadcasted_iota(jnp.int32, (BMo, BS), 0) + m * BMo
        qa = (rows == im_ref[...]).astype(jnp.bfloat16)
        qb = (rows == ip_ref[...]).astype(jnp.bfloat16)
        o_ref[...] += jnp.dot(qa, ya_ref[...],
                              preferred_element_type=jnp.float32)
        o_ref[...] += jnp.dot(qb, yb_ref[...],
                              preferred_element_type=jnp.float32)

    return pl.pallas_call(
        body,
        grid=(T // BMo, E * Cp // BS),
        out_shape=jax.ShapeDtypeStruct((T, D), jnp.float32),
        in_specs=[
            pl.BlockSpec((1, BS), lambda m, s: (0, s)),
            pl.BlockSpec((1, BS), lambda m, s: (0, s)),
            pl.BlockSpec((BS, D), lambda m, s: (s, 0)),
            pl.BlockSpec((BS, D), lambda m, s: (s, 0)),
        ],
        out_specs=pl.BlockSpec((BMo, D), lambda m, s: (m, 0)),
        compiler_params=pltpu.CompilerParams(
            vmem_limit_bytes=48 * 1024 * 1024),
    )(idx_mine2d, idx_peer2d, ya, yb)


NFB = F // BF


def _moe_fused(xg, W1, W2):

    def body(xloc_ref, xg_ref, w1_ref, w2_ref, ye_ref, yret_ref,
             xrecv, ystage, xs_send, xs_recv, ys_send, ys_recv):
        h = pl.program_id(0)
        j = pl.program_id(1)
        f = pl.program_id(2)

        def x_rdma(jj):
            return pltpu.make_async_remote_copy(
                src_ref=xg_ref.at[pl.ds((E + jj) * Cp, Cp)],
                dst_ref=xrecv.at[pl.ds(jj * Cp, Cp)],
                send_sem=xs_send.at[jj],
                recv_sem=xs_recv.at[jj],
                device_id=_peer(),
                device_id_type=pl.DeviceIdType.MESH,
            )

        def y_rdma(jj):
            return pltpu.make_async_remote_copy(
                src_ref=ystage.at[pl.ds(jj * Cp, Cp)],
                dst_ref=yret_ref.at[pl.ds(jj * Cp, Cp)],
                send_sem=ys_send.at[jj],
                recv_sem=ys_recv.at[jj],
                device_id=_peer(),
                device_id_type=pl.DeviceIdType.MESH,
            )

        is_first = jnp.logical_and(
            h == 0, jnp.logical_and(j == 0, f == 0))

        @pl.when(is_first)
        def _():
            _neighbor_barrier()
            for jj in range(E):
                x_rdma(jj).start()

        @pl.when(f == 0)
        def _():
            ye_ref[...] = jnp.zeros_like(ye_ref)

        @pl.when(jnp.logical_and(h == 1, f == 0))
        def _():
            x_rdma(j).wait_recv()

        w1 = w1_ref[0].astype(jnp.bfloat16)
        w2 = w2_ref[0].astype(jnp.bfloat16)

        @pl.when(h == 0)
        def _():
            hm = jnp.maximum(
                jnp.dot(xloc_ref[...], w1,
                        preferred_element_type=jnp.float32), 0.0)
            ye_ref[...] += jnp.dot(hm.astype(jnp.bfloat16), w2,
                                   preferred_element_type=jnp.float32)

        @pl.when(h == 1)
        def _():
            xb = xrecv[pl.ds(j * Cp, Cp), :]
            hm = jnp.maximum(
                jnp.dot(xb, w1, preferred_element_type=jnp.float32), 0.0)
            ye_ref[...] += jnp.dot(hm.astype(jnp.bfloat16), w2,
                                   preferred_element_type=jnp.float32)

        @pl.when(jnp.logical_and(h == 1, f == NFB - 1))
        def _():
            ystage[pl.ds(j * Cp, Cp), :] = ye_ref[...].astype(jnp.bfloat16)
            y_rdma(j).start()

        is_last = jnp.logical_and(
            h == 1, jnp.logical_and(j == E - 1, f == NFB - 1))

        @pl.when(is_last)
        def _():
            for jj in range(E):
                x_rdma(jj).wait_send()
                y_rdma(jj).wait_send()
                y_rdma(jj).wait_recv()

    return pl.pallas_call(
        body,
        grid=(2, E, NFB),
        out_shape=[
            jax.ShapeDtypeStruct((2 * E * Cp, D), jnp.float32),
            jax.ShapeDtypeStruct((E * Cp, D), jnp.bfloat16),
        ],
        in_specs=[
            pl.BlockSpec((Cp, D), lambda h, j, f: (j, 0)),
            pl.BlockSpec(memory_space=pl.ANY),
            pl.BlockSpec((1, D, BF), lambda h, j, f: (j, 0, f)),
            pl.BlockSpec((1, BF, D), lambda h, j, f: (j, f, 0)),
        ],
        out_specs=[
            pl.BlockSpec((Cp, D), lambda h, j, f: (h * E + j, 0)),
            pl.BlockSpec(memory_space=pl.ANY),
        ],
        scratch_shapes=[
            pltpu.VMEM((E * Cp, D), jnp.bfloat16),
            pltpu.VMEM((E * Cp, D), jnp.bfloat16),
            pltpu.SemaphoreType.DMA((E,)),
            pltpu.SemaphoreType.DMA((E,)),
            pltpu.SemaphoreType.DMA((E,)),
            pltpu.SemaphoreType.DMA((E,)),
        ],
        compiler_params=pltpu.CompilerParams(
            collective_id=0,
            vmem_limit_bytes=56 * 1024 * 1024),
    )(xg, xg, W1, W2)


def kernel(x, assign, W1, W2):
    my_y = lax.axis_index("y")
    e0 = E * my_y
    p0 = E - e0

    iota = jnp.arange(T, dtype=jnp.int32)
    prio = jnp.where(
        assign[None, :] == jnp.arange(8, dtype=jnp.int32)[:, None],
        iota[None, :], T)
    idx8 = jnp.sort(prio, axis=1)[:, :Cp]
    idx = jnp.concatenate([
        lax.dynamic_slice(idx8, (e0, 0), (E, Cp)),
        lax.dynamic_slice(idx8, (p0, 0), (E, Cp)),
    ]).reshape(-1)

    xg = _route_gather(jnp.minimum(idx, T - 1).reshape(8 * Cp, 1), x)

    ye_all, y_ret = _moe_fused(xg, W1, W2)

    return _unroute(
        idx[:E * Cp].reshape(1, E * Cp),
        idx[E * Cp:].reshape(1, E * Cp),
        ye_all[:E * Cp].astype(jnp.bfloat16),
        y_ret,
    )
